# baseline (speedup 1.0000x reference)
import functools

import jax
import jax.numpy as jnp
from jax.experimental import pallas as pl
from jax.experimental.pallas import tpu as pltpu

_BB = 16
_CB = 2


def _chunk_out(x, c, c2, w):
    cb_n, d = x.shape
    k = c.shape[0]
    n = cb_n // _CB

    x2 = jnp.sum(x * x, axis=1, keepdims=True)
    xcn = jax.lax.dot_general(
        x * -2.0, c, (((1,), (1,)), ((), ())),
        preferred_element_type=jnp.float32)
    e = (x2 + c2) + xcn

    emin = jnp.min(e, axis=1, keepdims=True)
    min_d = jnp.sqrt(jnp.maximum(emin, 0.0))[:, 0]

    md = min_d.reshape(_CB, n)
    mean = jnp.mean(md, axis=1, keepdims=True)
    std = jnp.sqrt(jnp.sum((md - mean) ** 2, axis=1, keepdims=True) / (n - 1))
    mask = (md < mean + std).astype(jnp.float32).reshape(cb_n)

    contrib = jnp.where(e == emin, mask[:, None], 0.0)
    hist = jnp.sum(contrib.reshape(_CB, n, k), axis=1)

    asmk = w * hist
    norm = jnp.sqrt(jnp.sum(asmk * asmk, axis=1, keepdims=True))
    return asmk / jnp.maximum(norm, 1e-12)


def _asmk_kernel(x_ref, c_ref, w_ref, out_ref):
    bb, n, d = x_ref.shape
    k = c_ref.shape[0]
    c = c_ref[...]
    c2 = jnp.sum(c * c, axis=1)[None, :]
    w = w_ref[...]
    for j in range(0, bb, _CB):
        x = x_ref[j:j + _CB].reshape(_CB * n, d)
        out_ref[j:j + _CB] = _chunk_out(x, c, c2, w).reshape(_CB, 1, k)


@functools.partial(jax.jit, static_argnames=())
def kernel(x, centroids, weights):
    B, N, D = x.shape
    K = centroids.shape[0]
    w2d = weights.reshape(1, K)
    return pl.pallas_call(
        _asmk_kernel,
        grid=(B // _BB,),
        in_specs=[
            pl.BlockSpec((_BB, N, D), lambda b: (b, 0, 0)),
            pl.BlockSpec((K, D), lambda b: (0, 0)),
            pl.BlockSpec((1, K), lambda b: (0, 0)),
        ],
        out_specs=pl.BlockSpec((_BB, 1, K), lambda b: (b, 0, 0)),
        out_shape=jax.ShapeDtypeStruct((B, 1, K), x.dtype),
        compiler_params=pltpu.CompilerParams(
            dimension_semantics=("arbitrary",)),
    )(x, centroids, w2d).reshape(B, K)


# speedup vs baseline: 1.0033x; 1.0033x over previous
import functools

import jax
import jax.numpy as jnp
from jax.experimental import pallas as pl
from jax.experimental.pallas import tpu as pltpu

_BB = 8
_CB = 2


def _chunk_out(x, c, c2, w):
    cb_n, d = x.shape
    k = c.shape[0]
    n = cb_n // _CB

    x2 = jnp.sum(x * x, axis=1, keepdims=True)
    xcn = jax.lax.dot_general(
        x * -2.0, c, (((1,), (1,)), ((), ())),
        preferred_element_type=jnp.float32)
    e = (x2 + c2) + xcn

    emin = jnp.min(e, axis=1, keepdims=True)
    min_d = jnp.sqrt(jnp.maximum(emin, 0.0))[:, 0]

    md = min_d.reshape(_CB, n)
    mean = jnp.mean(md, axis=1, keepdims=True)
    std = jnp.sqrt(jnp.sum((md - mean) ** 2, axis=1, keepdims=True) / (n - 1))
    mask = (md < mean + std).astype(jnp.float32).reshape(cb_n)

    contrib = jnp.where(e == emin, mask[:, None], 0.0)
    hist = jnp.sum(contrib.reshape(_CB, n, k), axis=1)

    asmk = w * hist
    norm = jnp.sqrt(jnp.sum(asmk * asmk, axis=1, keepdims=True))
    return asmk / jnp.maximum(norm, 1e-12)


def _asmk_kernel(x_ref, c_ref, w_ref, out_ref):
    bb, n, d = x_ref.shape
    k = c_ref.shape[0]
    c = c_ref[...]
    c2 = jnp.sum(c * c, axis=1)[None, :]
    w = w_ref[...]
    for j in range(0, bb, _CB):
        x = x_ref[j:j + _CB].reshape(_CB * n, d)
        out_ref[j:j + _CB] = _chunk_out(x, c, c2, w).reshape(_CB, 1, k)


@functools.partial(jax.jit, static_argnames=())
def kernel(x, centroids, weights):
    B, N, D = x.shape
    K = centroids.shape[0]
    w2d = weights.reshape(1, K)
    return pl.pallas_call(
        _asmk_kernel,
        grid=(B // _BB,),
        in_specs=[
            pl.BlockSpec((_BB, N, D), lambda b: (b, 0, 0)),
            pl.BlockSpec((K, D), lambda b: (0, 0)),
            pl.BlockSpec((1, K), lambda b: (0, 0)),
        ],
        out_specs=pl.BlockSpec((_BB, 1, K), lambda b: (b, 0, 0)),
        out_shape=jax.ShapeDtypeStruct((B, 1, K), x.dtype),
        compiler_params=pltpu.CompilerParams(
            dimension_semantics=("arbitrary",)),
    )(x, centroids, w2d).reshape(B, K)
